# baseline (speedup 1.0000x reference)
import jax
import jax.numpy as jnp
from jax.experimental import pallas as pl
from jax.experimental.pallas import tpu as pltpu

PLACEHOLDER_TOKEN = 265
B, N, D = 1024, 77, 768
ROWS = B * N
BLOCK_R = 1792


def _select_body(tok_ref, ph_ref, x_ref, o_ref):
    mask = tok_ref[...] == PLACEHOLDER_TOKEN
    o_ref[...] = jnp.where(mask, ph_ref[...], x_ref[...])


def kernel(tokenized_text, embedded_text, placeholder_embedding):
    tok = tokenized_text.reshape(ROWS, 1)
    x = embedded_text.reshape(ROWS, D)
    grid = (ROWS // BLOCK_R,)
    out = pl.pallas_call(
        _select_body,
        grid=grid,
        in_specs=[
            pl.BlockSpec((BLOCK_R, 1), lambda i: (i, 0)),
            pl.BlockSpec((1, D), lambda i: (0, 0)),
            pl.BlockSpec((BLOCK_R, D), lambda i: (i, 0)),
        ],
        out_specs=pl.BlockSpec((BLOCK_R, D), lambda i: (i, 0)),
        out_shape=jax.ShapeDtypeStruct((ROWS, D), jnp.float32),
    )(tok, placeholder_embedding, x)
    return out.reshape(B, N, D)


# speedup vs baseline: 1.6201x; 1.6201x over previous
import jax
import jax.numpy as jnp
from jax.experimental import pallas as pl
from jax.experimental.pallas import tpu as pltpu

PLACEHOLDER_TOKEN = 265
B, N, D = 1024, 77, 768
BLOCK_B = 16


def _select_body(tok_ref, ph_ref, x_ref, o_ref):
    mask = tok_ref[...] == PLACEHOLDER_TOKEN
    o_ref[...] = jnp.where(mask, ph_ref[...], x_ref[...])


def kernel(tokenized_text, embedded_text, placeholder_embedding):
    ph = placeholder_embedding.reshape(1, 1, D)
    tok = tokenized_text.reshape(B, N, 1)
    grid = (B // BLOCK_B,)
    out = pl.pallas_call(
        _select_body,
        grid=grid,
        in_specs=[
            pl.BlockSpec((BLOCK_B, N, 1), lambda i: (i, 0, 0)),
            pl.BlockSpec((1, 1, D), lambda i: (0, 0, 0)),
            pl.BlockSpec((BLOCK_B, N, D), lambda i: (i, 0, 0)),
        ],
        out_specs=pl.BlockSpec((BLOCK_B, N, D), lambda i: (i, 0, 0)),
        out_shape=jax.ShapeDtypeStruct((B, N, D), jnp.float32),
    )(tok, ph, embedded_text)
    return out
